# Initial kernel scaffold; baseline (speedup 1.0000x reference)
import functools

import jax
import jax.numpy as jnp
from jax import lax
from jax.experimental import pallas as pl
from jax.experimental.pallas import tpu as pltpu
from jax.experimental.pallas import tpu_sc as plsc

N = 10000
D = 128
O = 40
E = 320000

NC = 2
NS = 16
NW = NC * NS

CH = 100
ER = E // CH
RW = ER // NW
NPT = N // NS
DEGW = 16

_mesh = plsc.VectorSubcoreMesh(
    core_axis_name="c", subcore_axis_name="s", num_cores=NC, num_subcores=NS
)


@functools.partial(
    pl.kernel,
    out_type=jax.ShapeDtypeStruct((NC, N, DEGW), jnp.float32),
    mesh=_mesh,
    scratch_types=[
        pltpu.VMEM((RW, CH), jnp.int32),
        pltpu.VMEM((CH, DEGW), jnp.float32),
        pltpu.VMEM((NPT, DEGW), jnp.float32),
        pltpu.VMEM_SHARED((N, DEGW), jnp.float32),
    ],
)
def _deg_kernel(dst_hbm, out_hbm, dst_v, ones_v, zero_v, acc):
    c = lax.axis_index("c")
    s = lax.axis_index("s")
    wid = s * NC + c

    @pl.loop(0, NPT)
    def _(r):
        zero_v[r] = jnp.zeros((DEGW,), jnp.float32)

        @pl.when(r < CH)
        def _():
            ones_v[r] = jnp.ones((DEGW,), jnp.float32)

    pltpu.sync_copy(zero_v, acc.at[pl.ds(NPT * s, NPT)])
    plsc.subcore_barrier()

    pltpu.sync_copy(dst_hbm.at[pl.ds(wid * RW, RW)], dst_v)

    @pl.loop(0, RW)
    def _(r):
        pltpu.sync_copy(ones_v, acc.at[dst_v.at[r]], add=True)

    plsc.subcore_barrier()
    pltpu.sync_copy(acc.at[pl.ds(NPT * s, NPT)], out_hbm.at[c, pl.ds(NPT * s, NPT)])


@functools.partial(
    pl.kernel,
    out_type=jax.ShapeDtypeStruct((NC, N, D), jnp.float32),
    mesh=_mesh,
    scratch_types=[
        pltpu.VMEM((RW, CH), jnp.int32),
        pltpu.VMEM((RW, CH), jnp.int32),
        pltpu.VMEM((CH, D), jnp.float32),
        pltpu.VMEM((NPT // 5, D), jnp.float32),
        pltpu.VMEM_SHARED((N, D), jnp.float32),
        pltpu.SemaphoreType.DMA,
    ],
)
def _agg_kernel(table_hbm, src_hbm, dst_hbm, out_hbm,
                src_v, dst_v, rows_v, zero_v, acc, sem):
    c = lax.axis_index("c")
    s = lax.axis_index("s")
    wid = s * NC + c

    zr = NPT // 5

    @pl.loop(0, zr * (D // 16))
    def _(i):
        r = i // (D // 16)
        k = i % (D // 16)
        zero_v[r, pl.ds(k * 16, 16)] = jnp.zeros((16,), jnp.float32)

    for j in range(5):
        pltpu.sync_copy(zero_v, acc.at[pl.ds(NPT * s + zr * j, zr)])
    plsc.subcore_barrier()

    pltpu.sync_copy(src_hbm.at[pl.ds(wid * RW, RW)], src_v)
    pltpu.sync_copy(dst_hbm.at[pl.ds(wid * RW, RW)], dst_v)

    @pl.loop(0, RW)
    def _(r):
        pltpu.async_copy(table_hbm.at[src_v.at[r]], rows_v, sem).wait()
        pltpu.sync_copy(rows_v, acc.at[dst_v.at[r]], add=True)

    plsc.subcore_barrier()
    pltpu.sync_copy(acc.at[pl.ds(NPT * s, NPT)], out_hbm.at[c, pl.ds(NPT * s, NPT)])


BR = 2000
G = N // BR


def _mm_relu_body(x_ref, w_ref, b_ref, o_ref):
    o_ref[...] = jnp.maximum(
        jnp.dot(x_ref[...], w_ref[...], preferred_element_type=jnp.float32)
        + b_ref[...],
        0.0,
    )


def _mm_relu(x, w, b):
    return pl.pallas_call(
        _mm_relu_body,
        grid=(G,),
        in_specs=[
            pl.BlockSpec((BR, D), lambda i: (i, 0)),
            pl.BlockSpec((D, D), lambda i: (0, 0)),
            pl.BlockSpec((1, D), lambda i: (0, 0)),
        ],
        out_specs=pl.BlockSpec((BR, D), lambda i: (i, 0)),
        out_shape=jax.ShapeDtypeStruct((N, D), jnp.float32),
    )(x, w, b.reshape(1, D))


def _pre_body(h_ref, w_ref, d0_ref, d1_ref, s_ref, dis_ref):
    deg = d0_ref[...] + d1_ref[...] + 1.0
    dis = lax.rsqrt(deg)
    dis_ref[...] = dis
    s_ref[...] = (
        jnp.dot(h_ref[...], w_ref[...], preferred_element_type=jnp.float32) * dis
    )


def _pre(h, w, d0, d1):
    return pl.pallas_call(
        _pre_body,
        grid=(G,),
        in_specs=[
            pl.BlockSpec((BR, D), lambda i: (i, 0)),
            pl.BlockSpec((D, D), lambda i: (0, 0)),
            pl.BlockSpec((BR, 1), lambda i: (i, 0)),
            pl.BlockSpec((BR, 1), lambda i: (i, 0)),
        ],
        out_specs=[
            pl.BlockSpec((BR, D), lambda i: (i, 0)),
            pl.BlockSpec((BR, 1), lambda i: (i, 0)),
        ],
        out_shape=[
            jax.ShapeDtypeStruct((N, D), jnp.float32),
            jax.ShapeDtypeStruct((N, 1), jnp.float32),
        ],
    )(h, w, d0, d1)


def _post_body(p0_ref, p1_ref, s_ref, dis_ref, b_ref, t_ref, sums_ref):
    t = dis_ref[...] * (p0_ref[...] + p1_ref[...] + s_ref[...]) + b_ref[...]
    t_ref[...] = t
    st = jnp.sum(t)
    st2 = jnp.sum(t * t)
    col = lax.broadcasted_iota(jnp.int32, (1, D), 1)
    sums_ref[0] = jnp.where(col == 0, st, 0.0) + jnp.where(col == 1, st2, 0.0)


def _post(p0, p1, s, dis, b):
    return pl.pallas_call(
        _post_body,
        grid=(G,),
        in_specs=[
            pl.BlockSpec((BR, D), lambda i: (i, 0)),
            pl.BlockSpec((BR, D), lambda i: (i, 0)),
            pl.BlockSpec((BR, D), lambda i: (i, 0)),
            pl.BlockSpec((BR, 1), lambda i: (i, 0)),
            pl.BlockSpec((1, D), lambda i: (0, 0)),
        ],
        out_specs=[
            pl.BlockSpec((BR, D), lambda i: (i, 0)),
            pl.BlockSpec((1, 1, D), lambda i: (i, 0, 0)),
        ],
        out_shape=[
            jax.ShapeDtypeStruct((N, D), jnp.float32),
            jax.ShapeDtypeStruct((G, 1, D), jnp.float32),
        ],
    )(p0, p1, s, dis, b.reshape(1, D))


_M = float(N * D)
_EPS = 1e-5


def _ln_mm_body(t_ref, sums_ref, dis_ref, g_ref, be_ref, w_ref, o_ref):
    sums = sums_ref[...]
    mean = jnp.sum(sums[:, 0, 0]) / _M
    var = jnp.sum(sums[:, 0, 1]) / _M - mean * mean
    inv = lax.rsqrt(var + _EPS)
    h = jnp.maximum((t_ref[...] - mean) * inv * g_ref[...] + be_ref[...], 0.0)
    o_ref[...] = (
        jnp.dot(h, w_ref[...], preferred_element_type=jnp.float32) * dis_ref[...]
    )


def _ln_mm(t, sums, dis, g, be, w):
    return pl.pallas_call(
        _ln_mm_body,
        grid=(G,),
        in_specs=[
            pl.BlockSpec((BR, D), lambda i: (i, 0)),
            pl.BlockSpec((G, 1, D), lambda i: (0, 0, 0)),
            pl.BlockSpec((BR, 1), lambda i: (i, 0)),
            pl.BlockSpec((1, D), lambda i: (0, 0)),
            pl.BlockSpec((1, D), lambda i: (0, 0)),
            pl.BlockSpec((D, D), lambda i: (0, 0)),
        ],
        out_specs=pl.BlockSpec((BR, D), lambda i: (i, 0)),
        out_shape=jax.ShapeDtypeStruct((N, D), jnp.float32),
    )(t, sums, dis, g.reshape(1, D), be.reshape(1, D), w)


def _final_body(t_ref, sums_ref, g_ref, be_ref, w_ref, b_ref, o_ref):
    sums = sums_ref[...]
    mean = jnp.sum(sums[:, 0, 0]) / _M
    var = jnp.sum(sums[:, 0, 1]) / _M - mean * mean
    inv = lax.rsqrt(var + _EPS)
    h = jnp.maximum((t_ref[...] - mean) * inv * g_ref[...] + be_ref[...], 0.0)
    logits = (
        jnp.dot(h, w_ref[...], preferred_element_type=jnp.float32) + b_ref[...]
    )
    m = jnp.max(logits, axis=-1, keepdims=True)
    e = jnp.exp(logits - m)
    o_ref[...] = e / jnp.sum(e, axis=-1, keepdims=True)


def _final(t, sums, g, be, w, b):
    return pl.pallas_call(
        _final_body,
        grid=(G,),
        in_specs=[
            pl.BlockSpec((BR, D), lambda i: (i, 0)),
            pl.BlockSpec((G, 1, D), lambda i: (0, 0, 0)),
            pl.BlockSpec((1, D), lambda i: (0, 0)),
            pl.BlockSpec((1, D), lambda i: (0, 0)),
            pl.BlockSpec((D, O), lambda i: (0, 0)),
            pl.BlockSpec((1, O), lambda i: (0, 0)),
        ],
        out_specs=pl.BlockSpec((BR, O), lambda i: (i, 0)),
        out_shape=jax.ShapeDtypeStruct((N, O), jnp.float32),
    )(t, sums, g.reshape(1, D), be.reshape(1, D), w, b.reshape(1, O))


def kernel(x, edge_index, W_in, b_in, W1, b1, g1, be1, W2, b2, g2, be2,
           W_out, b_out):
    src2 = edge_index[0].reshape(ER, CH)
    dst2 = edge_index[1].reshape(ER, CH)

    degp = _deg_kernel(dst2)
    d0 = degp[0, :, 0:1]
    d1 = degp[1, :, 0:1]

    h1 = _mm_relu(x, W_in, b_in)
    scaled1, dis = _pre(h1, W1, d0, d1)

    pa = _agg_kernel(scaled1, src2, dst2)
    t1, s1 = _post(pa[0], pa[1], scaled1, dis, b1)
    scaled2 = _ln_mm(t1, s1, dis, g1, be1, W2)

    pb = _agg_kernel(scaled2, src2, dst2)
    t2, s2 = _post(pb[0], pb[1], scaled2, dis, b2)
    return _final(t2, s2, g2, be2, W_out, b_out)


# speedup vs baseline: 21.3452x; 21.3452x over previous
import jax
import jax.numpy as jnp
from jax import lax
from jax.experimental import pallas as pl
from jax.experimental.pallas import tpu as pltpu
from jax.experimental.pallas import tpu_sc as plsc

N = 10000
D = 128
O = 40
E = 320000

NC = 2
NS = 16
NW = NC * NS

NP = 10240
CH = 128
EP = 327680
ER = EP // CH
RW = ER // NW
NPT = NP // NS
EPW = EP // NW

_mesh = plsc.VectorSubcoreMesh(
    core_axis_name="c", subcore_axis_name="s", num_cores=NC, num_subcores=NS
)


_DEG_OUT = jax.ShapeDtypeStruct((NW * NP,), jnp.float32)
_DEG_SCRATCH = [
    pltpu.VMEM((EPW,), jnp.int32),
    pltpu.VMEM((NP,), jnp.float32),
]


def _deg_body(dst_hbm, out_hbm, dst_v, hist):
    c = lax.axis_index("c")
    s = lax.axis_index("s")
    wid = s * NC + c

    @pl.loop(0, NP // 16)
    def _(i):
        hist[pl.ds(i * 16, 16)] = jnp.zeros((16,), jnp.float32)

    pltpu.sync_copy(dst_hbm.at[pl.ds(wid * EPW, EPW)], dst_v)

    ones16 = jnp.ones((16,), jnp.float32)

    @pl.loop(0, EPW // 16)
    def _(i):
        d16 = dst_v[pl.ds(i * 16, 16)]
        plsc.addupdate_scatter(hist, [d16], ones16)

    pltpu.sync_copy(hist, out_hbm.at[pl.ds(wid * NP, NP)])


_deg_kernel = pl.kernel(
    _deg_body, out_type=_DEG_OUT, mesh=_mesh, scratch_types=_DEG_SCRATCH,
    compiler_params=pltpu.CompilerParams(needs_layout_passes=False),
)


_AGG_OUT = jax.ShapeDtypeStruct((NC, NP, D), jnp.float32)
_AGG_SCRATCH = [
    pltpu.VMEM((RW, CH), jnp.int32),
    pltpu.VMEM((RW, CH), jnp.int32),
    pltpu.VMEM((CH, D), jnp.float32),
    pltpu.VMEM((NPT // 8, D), jnp.float32),
    pltpu.VMEM_SHARED((NP, D), jnp.float32),
    pltpu.SemaphoreType.DMA,
]


def _agg_body(table_hbm, src_hbm, dst_hbm, out_hbm,
              src_v, dst_v, rows_v, zero_v, acc, sem):
    c = lax.axis_index("c")
    s = lax.axis_index("s")
    wid = s * NC + c

    zr = NPT // 8

    @pl.loop(0, zr)
    def _(r):
        for k in range(D // 16):
            zero_v[r, pl.ds(k * 16, 16)] = jnp.zeros((16,), jnp.float32)

    for j in range(8):
        pltpu.sync_copy(zero_v, acc.at[pl.ds(NPT * s + zr * j, zr)])
    plsc.subcore_barrier()

    pltpu.sync_copy(src_hbm.at[pl.ds(wid * RW, RW)], src_v)
    pltpu.sync_copy(dst_hbm.at[pl.ds(wid * RW, RW)], dst_v)

    @pl.loop(0, RW)
    def _(r):
        pltpu.async_copy(table_hbm.at[src_v.at[r]], rows_v, sem).wait()
        pltpu.sync_copy(rows_v, acc.at[dst_v.at[r]], add=True)

    plsc.subcore_barrier()
    pltpu.sync_copy(acc.at[pl.ds(NPT * s, NPT)], out_hbm.at[c, pl.ds(NPT * s, NPT)])


_agg_kernel = pl.kernel(
    _agg_body, out_type=_AGG_OUT, mesh=_mesh, scratch_types=_AGG_SCRATCH
)


BR = 2000
G = N // BR


def _sumhist_body(h_ref, o_ref):
    o_ref[...] = jnp.sum(h_ref[...], axis=0, keepdims=True) + 1.0


def _sumhist(hists):
    return pl.pallas_call(
        _sumhist_body,
        in_specs=[pl.BlockSpec((NW, NP), lambda: (0, 0))],
        out_specs=pl.BlockSpec((1, NP), lambda: (0, 0)),
        out_shape=jax.ShapeDtypeStruct((1, NP), jnp.float32),
    )(hists)


def _mm_relu_body(x_ref, w_ref, b_ref, o_ref):
    o_ref[...] = jnp.maximum(
        jnp.dot(x_ref[...], w_ref[...], preferred_element_type=jnp.float32)
        + b_ref[...],
        0.0,
    )


def _mm_relu(x, w, b):
    return pl.pallas_call(
        _mm_relu_body,
        grid=(G,),
        in_specs=[
            pl.BlockSpec((BR, D), lambda i: (i, 0)),
            pl.BlockSpec((D, D), lambda i: (0, 0)),
            pl.BlockSpec((1, D), lambda i: (0, 0)),
        ],
        out_specs=pl.BlockSpec((BR, D), lambda i: (i, 0)),
        out_shape=jax.ShapeDtypeStruct((N, D), jnp.float32),
    )(x, w, b.reshape(1, D))


def _pre_body(h_ref, w_ref, d_ref, s_ref, dis_ref):
    dis = lax.rsqrt(d_ref[...])
    dis_ref[...] = dis
    s_ref[...] = (
        jnp.dot(h_ref[...], w_ref[...], preferred_element_type=jnp.float32) * dis
    )


def _pre(h, w, d):
    return pl.pallas_call(
        _pre_body,
        grid=(G,),
        in_specs=[
            pl.BlockSpec((BR, D), lambda i: (i, 0)),
            pl.BlockSpec((D, D), lambda i: (0, 0)),
            pl.BlockSpec((BR, 1), lambda i: (i, 0)),
        ],
        out_specs=[
            pl.BlockSpec((BR, D), lambda i: (i, 0)),
            pl.BlockSpec((BR, 1), lambda i: (i, 0)),
        ],
        out_shape=[
            jax.ShapeDtypeStruct((N, D), jnp.float32),
            jax.ShapeDtypeStruct((N, 1), jnp.float32),
        ],
    )(h, w, d)


def _post_body(p0_ref, p1_ref, s_ref, dis_ref, b_ref, t_ref, sums_ref):
    t = dis_ref[...] * (p0_ref[...] + p1_ref[...] + s_ref[...]) + b_ref[...]
    t_ref[...] = t
    st = jnp.sum(t)
    st2 = jnp.sum(t * t)
    col = lax.broadcasted_iota(jnp.int32, (1, D), 1)
    sums_ref[0] = jnp.where(col == 0, st, 0.0) + jnp.where(col == 1, st2, 0.0)


def _post(p0, p1, s, dis, b):
    return pl.pallas_call(
        _post_body,
        grid=(G,),
        in_specs=[
            pl.BlockSpec((BR, D), lambda i: (i, 0)),
            pl.BlockSpec((BR, D), lambda i: (i, 0)),
            pl.BlockSpec((BR, D), lambda i: (i, 0)),
            pl.BlockSpec((BR, 1), lambda i: (i, 0)),
            pl.BlockSpec((1, D), lambda i: (0, 0)),
        ],
        out_specs=[
            pl.BlockSpec((BR, D), lambda i: (i, 0)),
            pl.BlockSpec((1, 1, D), lambda i: (i, 0, 0)),
        ],
        out_shape=[
            jax.ShapeDtypeStruct((N, D), jnp.float32),
            jax.ShapeDtypeStruct((G, 1, D), jnp.float32),
        ],
    )(p0, p1, s, dis, b.reshape(1, D))


_M = float(N * D)
_EPS = 1e-5


def _ln_mm_body(t_ref, sums_ref, dis_ref, g_ref, be_ref, w_ref, o_ref):
    sums = sums_ref[...]
    mean = jnp.sum(sums[:, 0, 0]) / _M
    var = jnp.sum(sums[:, 0, 1]) / _M - mean * mean
    inv = lax.rsqrt(var + _EPS)
    h = jnp.maximum((t_ref[...] - mean) * inv * g_ref[...] + be_ref[...], 0.0)
    o_ref[...] = (
        jnp.dot(h, w_ref[...], preferred_element_type=jnp.float32) * dis_ref[...]
    )


def _ln_mm(t, sums, dis, g, be, w):
    return pl.pallas_call(
        _ln_mm_body,
        grid=(G,),
        in_specs=[
            pl.BlockSpec((BR, D), lambda i: (i, 0)),
            pl.BlockSpec((G, 1, D), lambda i: (0, 0, 0)),
            pl.BlockSpec((BR, 1), lambda i: (i, 0)),
            pl.BlockSpec((1, D), lambda i: (0, 0)),
            pl.BlockSpec((1, D), lambda i: (0, 0)),
            pl.BlockSpec((D, D), lambda i: (0, 0)),
        ],
        out_specs=pl.BlockSpec((BR, D), lambda i: (i, 0)),
        out_shape=jax.ShapeDtypeStruct((N, D), jnp.float32),
    )(t, sums, dis, g.reshape(1, D), be.reshape(1, D), w)


def _final_body(t_ref, sums_ref, g_ref, be_ref, w_ref, b_ref, o_ref):
    sums = sums_ref[...]
    mean = jnp.sum(sums[:, 0, 0]) / _M
    var = jnp.sum(sums[:, 0, 1]) / _M - mean * mean
    inv = lax.rsqrt(var + _EPS)
    h = jnp.maximum((t_ref[...] - mean) * inv * g_ref[...] + be_ref[...], 0.0)
    logits = (
        jnp.dot(h, w_ref[...], preferred_element_type=jnp.float32) + b_ref[...]
    )
    m = jnp.max(logits, axis=-1, keepdims=True)
    e = jnp.exp(logits - m)
    o_ref[...] = e / jnp.sum(e, axis=-1, keepdims=True)


def _final(t, sums, g, be, w, b):
    return pl.pallas_call(
        _final_body,
        grid=(G,),
        in_specs=[
            pl.BlockSpec((BR, D), lambda i: (i, 0)),
            pl.BlockSpec((G, 1, D), lambda i: (0, 0, 0)),
            pl.BlockSpec((1, D), lambda i: (0, 0)),
            pl.BlockSpec((1, D), lambda i: (0, 0)),
            pl.BlockSpec((D, O), lambda i: (0, 0)),
            pl.BlockSpec((1, O), lambda i: (0, 0)),
        ],
        out_specs=pl.BlockSpec((BR, O), lambda i: (i, 0)),
        out_shape=jax.ShapeDtypeStruct((N, O), jnp.float32),
    )(t, sums, g.reshape(1, D), be.reshape(1, D), w, b.reshape(1, O))


def kernel(x, edge_index, W_in, b_in, W1, b1, g1, be1, W2, b2, g2, be2,
           W_out, b_out):
    npad = EP - E
    pad_src = (jnp.arange(npad, dtype=jnp.int32) * 991) % N
    pad_dst = N + (jnp.arange(npad, dtype=jnp.int32) % (NP - N))
    src1 = jnp.concatenate([edge_index[0], pad_src])
    dst1 = jnp.concatenate([edge_index[1], pad_dst])
    src2 = src1.reshape(ER, CH)
    dst2 = dst1.reshape(ER, CH)

    hists = _deg_kernel(dst1).reshape(NW, NP)
    degv = _sumhist(hists)
    d = degv.reshape(NP, 1)[:N]

    h1 = _mm_relu(x, W_in, b_in)
    scaled1, dis = _pre(h1, W1, d)

    pa = _agg_kernel(scaled1, src2, dst2)
    t1, s1 = _post(pa[0, :N], pa[1, :N], scaled1, dis, b1)
    scaled2 = _ln_mm(t1, s1, dis, g1, be1, W2)

    pb = _agg_kernel(scaled2, src2, dst2)
    t2, s2 = _post(pb[0, :N], pb[1, :N], scaled2, dis, b2)
    return _final(t2, s2, g2, be2, W_out, b_out)


# speedup vs baseline: 26.3634x; 1.2351x over previous
import jax
import jax.numpy as jnp
from jax import lax
from jax.experimental import pallas as pl
from jax.experimental.pallas import tpu as pltpu
from jax.experimental.pallas import tpu_sc as plsc

N = 10000
D = 128
O = 40
E = 320000

NC = 2
NS = 16
NW = NC * NS

NP = 10240
CH = 128
EP = 327680
ER = EP // CH
RW = ER // NW
NPT = NP // NS
EPW = EP // NW

_mesh = plsc.VectorSubcoreMesh(
    core_axis_name="c", subcore_axis_name="s", num_cores=NC, num_subcores=NS
)


_DEG_OUT = jax.ShapeDtypeStruct((NW * NP,), jnp.float32)
_DEG_SCRATCH = [
    pltpu.VMEM((EPW,), jnp.int32),
    pltpu.VMEM((NP,), jnp.float32),
]


def _deg_body(dst_hbm, out_hbm, dst_v, hist):
    c = lax.axis_index("c")
    s = lax.axis_index("s")
    wid = s * NC + c

    @pl.loop(0, NP // 16)
    def _(i):
        hist[pl.ds(i * 16, 16)] = jnp.zeros((16,), jnp.float32)

    pltpu.sync_copy(dst_hbm.at[pl.ds(wid * EPW, EPW)], dst_v)

    ones16 = jnp.ones((16,), jnp.float32)

    @pl.loop(0, EPW // 16)
    def _(i):
        d16 = dst_v[pl.ds(i * 16, 16)]
        plsc.addupdate_scatter(hist, [d16], ones16)

    pltpu.sync_copy(hist, out_hbm.at[pl.ds(wid * NP, NP)])


_deg_kernel = pl.kernel(
    _deg_body, out_type=_DEG_OUT, mesh=_mesh, scratch_types=_DEG_SCRATCH,
    compiler_params=pltpu.CompilerParams(needs_layout_passes=False),
)


_AGG_OUT = jax.ShapeDtypeStruct((NC, NP, D), jnp.float32)
_HRW = RW // 2
_AGG_SCRATCH = [
    pltpu.VMEM((_HRW, CH), jnp.int32),
    pltpu.VMEM((_HRW, CH), jnp.int32),
    pltpu.VMEM((CH, D), jnp.float32),
    pltpu.VMEM((CH, D), jnp.float32),
    pltpu.VMEM_SHARED((NP, D), jnp.float32),
    pltpu.SemaphoreType.DMA,
    pltpu.SemaphoreType.DMA,
]


def _agg_body(table_hbm, src_hbm, dst_hbm, out_hbm,
              src_v, dst_v, rows0, rows1, acc, sem0, sem1):
    c = lax.axis_index("c")
    s = lax.axis_index("s")
    wid = s * NC + c

    @pl.loop(0, CH)
    def _(r):
        for k in range(D // 16):
            rows0[r, pl.ds(k * 16, 16)] = jnp.zeros((16,), jnp.float32)

    for j in range(NPT // CH):
        pltpu.sync_copy(rows0, acc.at[pl.ds(NPT * s + CH * j, CH)])
    plsc.subcore_barrier()

    for ph in range(2):
        base = wid * RW + ph * _HRW
        pltpu.sync_copy(src_hbm.at[pl.ds(base, _HRW)], src_v)
        pltpu.sync_copy(dst_hbm.at[pl.ds(base, _HRW)], dst_v)
        pltpu.async_copy(table_hbm.at[src_v.at[0]], rows0, sem0)

        @pl.loop(0, _HRW // 2)
        def _(k):
            pltpu.make_async_copy(table_hbm.at[src_v.at[0]], rows0, sem0).wait()
            pltpu.async_copy(table_hbm.at[src_v.at[2 * k + 1]], rows1, sem1)
            pltpu.sync_copy(rows0, acc.at[dst_v.at[2 * k]], add=True)
            pltpu.make_async_copy(table_hbm.at[src_v.at[0]], rows1, sem1).wait()

            @pl.when(k < _HRW // 2 - 1)
            def _():
                pltpu.async_copy(table_hbm.at[src_v.at[2 * k + 2]], rows0, sem0)

            pltpu.sync_copy(rows1, acc.at[dst_v.at[2 * k + 1]], add=True)

    plsc.subcore_barrier()
    pltpu.sync_copy(acc.at[pl.ds(NPT * s, NPT)], out_hbm.at[c, pl.ds(NPT * s, NPT)])


_agg_kernel = pl.kernel(
    _agg_body, out_type=_AGG_OUT, mesh=_mesh, scratch_types=_AGG_SCRATCH
)


BR = 2000
G = N // BR


def _sumhist_body(h_ref, o_ref):
    o_ref[...] = jnp.sum(h_ref[...], axis=0, keepdims=True) + 1.0


def _sumhist(hists):
    return pl.pallas_call(
        _sumhist_body,
        in_specs=[pl.BlockSpec((NW, NP), lambda: (0, 0))],
        out_specs=pl.BlockSpec((1, NP), lambda: (0, 0)),
        out_shape=jax.ShapeDtypeStruct((1, NP), jnp.float32),
    )(hists)


def _mm_relu_body(x_ref, w_ref, b_ref, o_ref):
    o_ref[...] = jnp.maximum(
        jnp.dot(x_ref[...], w_ref[...], preferred_element_type=jnp.float32)
        + b_ref[...],
        0.0,
    )


def _mm_relu(x, w, b):
    return pl.pallas_call(
        _mm_relu_body,
        grid=(G,),
        in_specs=[
            pl.BlockSpec((BR, D), lambda i: (i, 0)),
            pl.BlockSpec((D, D), lambda i: (0, 0)),
            pl.BlockSpec((1, D), lambda i: (0, 0)),
        ],
        out_specs=pl.BlockSpec((BR, D), lambda i: (i, 0)),
        out_shape=jax.ShapeDtypeStruct((N, D), jnp.float32),
    )(x, w, b.reshape(1, D))


def _pre_body(h_ref, w_ref, d_ref, s_ref, dis_ref):
    dis = lax.rsqrt(d_ref[...])
    dis_ref[...] = dis
    s_ref[...] = (
        jnp.dot(h_ref[...], w_ref[...], preferred_element_type=jnp.float32) * dis
    )


def _pre(h, w, d):
    return pl.pallas_call(
        _pre_body,
        grid=(G,),
        in_specs=[
            pl.BlockSpec((BR, D), lambda i: (i, 0)),
            pl.BlockSpec((D, D), lambda i: (0, 0)),
            pl.BlockSpec((BR, 1), lambda i: (i, 0)),
        ],
        out_specs=[
            pl.BlockSpec((BR, D), lambda i: (i, 0)),
            pl.BlockSpec((BR, 1), lambda i: (i, 0)),
        ],
        out_shape=[
            jax.ShapeDtypeStruct((N, D), jnp.float32),
            jax.ShapeDtypeStruct((N, 1), jnp.float32),
        ],
    )(h, w, d)


def _post_body(p0_ref, p1_ref, s_ref, dis_ref, b_ref, t_ref, sums_ref):
    t = dis_ref[...] * (p0_ref[...] + p1_ref[...] + s_ref[...]) + b_ref[...]
    t_ref[...] = t
    st = jnp.sum(t)
    st2 = jnp.sum(t * t)
    col = lax.broadcasted_iota(jnp.int32, (1, D), 1)
    sums_ref[0] = jnp.where(col == 0, st, 0.0) + jnp.where(col == 1, st2, 0.0)


def _post(p0, p1, s, dis, b):
    return pl.pallas_call(
        _post_body,
        grid=(G,),
        in_specs=[
            pl.BlockSpec((BR, D), lambda i: (i, 0)),
            pl.BlockSpec((BR, D), lambda i: (i, 0)),
            pl.BlockSpec((BR, D), lambda i: (i, 0)),
            pl.BlockSpec((BR, 1), lambda i: (i, 0)),
            pl.BlockSpec((1, D), lambda i: (0, 0)),
        ],
        out_specs=[
            pl.BlockSpec((BR, D), lambda i: (i, 0)),
            pl.BlockSpec((1, 1, D), lambda i: (i, 0, 0)),
        ],
        out_shape=[
            jax.ShapeDtypeStruct((N, D), jnp.float32),
            jax.ShapeDtypeStruct((G, 1, D), jnp.float32),
        ],
    )(p0, p1, s, dis, b.reshape(1, D))


_M = float(N * D)
_EPS = 1e-5


def _ln_mm_body(t_ref, sums_ref, dis_ref, g_ref, be_ref, w_ref, o_ref):
    sums = sums_ref[...]
    mean = jnp.sum(sums[:, 0, 0]) / _M
    var = jnp.sum(sums[:, 0, 1]) / _M - mean * mean
    inv = lax.rsqrt(var + _EPS)
    h = jnp.maximum((t_ref[...] - mean) * inv * g_ref[...] + be_ref[...], 0.0)
    o_ref[...] = (
        jnp.dot(h, w_ref[...], preferred_element_type=jnp.float32) * dis_ref[...]
    )


def _ln_mm(t, sums, dis, g, be, w):
    return pl.pallas_call(
        _ln_mm_body,
        grid=(G,),
        in_specs=[
            pl.BlockSpec((BR, D), lambda i: (i, 0)),
            pl.BlockSpec((G, 1, D), lambda i: (0, 0, 0)),
            pl.BlockSpec((BR, 1), lambda i: (i, 0)),
            pl.BlockSpec((1, D), lambda i: (0, 0)),
            pl.BlockSpec((1, D), lambda i: (0, 0)),
            pl.BlockSpec((D, D), lambda i: (0, 0)),
        ],
        out_specs=pl.BlockSpec((BR, D), lambda i: (i, 0)),
        out_shape=jax.ShapeDtypeStruct((N, D), jnp.float32),
    )(t, sums, dis, g.reshape(1, D), be.reshape(1, D), w)


def _final_body(t_ref, sums_ref, g_ref, be_ref, w_ref, b_ref, o_ref):
    sums = sums_ref[...]
    mean = jnp.sum(sums[:, 0, 0]) / _M
    var = jnp.sum(sums[:, 0, 1]) / _M - mean * mean
    inv = lax.rsqrt(var + _EPS)
    h = jnp.maximum((t_ref[...] - mean) * inv * g_ref[...] + be_ref[...], 0.0)
    logits = (
        jnp.dot(h, w_ref[...], preferred_element_type=jnp.float32) + b_ref[...]
    )
    m = jnp.max(logits, axis=-1, keepdims=True)
    e = jnp.exp(logits - m)
    o_ref[...] = e / jnp.sum(e, axis=-1, keepdims=True)


def _final(t, sums, g, be, w, b):
    return pl.pallas_call(
        _final_body,
        grid=(G,),
        in_specs=[
            pl.BlockSpec((BR, D), lambda i: (i, 0)),
            pl.BlockSpec((G, 1, D), lambda i: (0, 0, 0)),
            pl.BlockSpec((1, D), lambda i: (0, 0)),
            pl.BlockSpec((1, D), lambda i: (0, 0)),
            pl.BlockSpec((D, O), lambda i: (0, 0)),
            pl.BlockSpec((1, O), lambda i: (0, 0)),
        ],
        out_specs=pl.BlockSpec((BR, O), lambda i: (i, 0)),
        out_shape=jax.ShapeDtypeStruct((N, O), jnp.float32),
    )(t, sums, g.reshape(1, D), be.reshape(1, D), w, b.reshape(1, O))


def kernel(x, edge_index, W_in, b_in, W1, b1, g1, be1, W2, b2, g2, be2,
           W_out, b_out):
    npad = EP - E
    pad_src = (jnp.arange(npad, dtype=jnp.int32) * 991) % N
    pad_dst = N + (jnp.arange(npad, dtype=jnp.int32) % (NP - N))
    src1 = jnp.concatenate([edge_index[0], pad_src])
    dst1 = jnp.concatenate([edge_index[1], pad_dst])
    src2 = src1.reshape(ER, CH)
    dst2 = dst1.reshape(ER, CH)

    hists = _deg_kernel(dst1).reshape(NW, NP)
    degv = _sumhist(hists)
    d = degv.reshape(NP, 1)[:N]

    h1 = _mm_relu(x, W_in, b_in)
    scaled1, dis = _pre(h1, W1, d)

    pa = _agg_kernel(scaled1, src2, dst2)
    t1, s1 = _post(pa[0, :N], pa[1, :N], scaled1, dis, b1)
    scaled2 = _ln_mm(t1, s1, dis, g1, be1, W2)

    pb = _agg_kernel(scaled2, src2, dst2)
    t2, s2 = _post(pb[0, :N], pb[1, :N], scaled2, dis, b2)
    return _final(t2, s2, g2, be2, W_out, b_out)


# speedup vs baseline: 27.4520x; 1.0413x over previous
import jax
import jax.numpy as jnp
from jax import lax
from jax.experimental import pallas as pl
from jax.experimental.pallas import tpu as pltpu
from jax.experimental.pallas import tpu_sc as plsc

N = 10000
D = 128
O = 40
E = 320000

NC = 2
NS = 16
NW = NC * NS

NP = 10240
CH = 128
EP = 327680
ER = EP // CH
RW = ER // NW
NPT = NP // NS
EPW = EP // NW

_mesh = plsc.VectorSubcoreMesh(
    core_axis_name="c", subcore_axis_name="s", num_cores=NC, num_subcores=NS
)


_DEG_OUT = jax.ShapeDtypeStruct((NW * NP,), jnp.float32)
_DEG_SCRATCH = [
    pltpu.VMEM((EPW,), jnp.int32),
    pltpu.VMEM((NP,), jnp.float32),
]


def _deg_body(dst_hbm, out_hbm, dst_v, hist):
    c = lax.axis_index("c")
    s = lax.axis_index("s")
    wid = s * NC + c

    @pl.loop(0, NP // 16)
    def _(i):
        hist[pl.ds(i * 16, 16)] = jnp.zeros((16,), jnp.float32)

    pltpu.sync_copy(dst_hbm.at[pl.ds(wid * EPW, EPW)], dst_v)

    ones16 = jnp.ones((16,), jnp.float32)

    @pl.loop(0, EPW // 16)
    def _(i):
        d16 = dst_v[pl.ds(i * 16, 16)]
        plsc.addupdate_scatter(hist, [d16], ones16)

    pltpu.sync_copy(hist, out_hbm.at[pl.ds(wid * NP, NP)])


_deg_kernel = pl.kernel(
    _deg_body, out_type=_DEG_OUT, mesh=_mesh, scratch_types=_DEG_SCRATCH,
    compiler_params=pltpu.CompilerParams(needs_layout_passes=False),
)


_AGG_OUT = jax.ShapeDtypeStruct((NC, NP, D), jnp.float32)
_HRW = RW // 2
_AGG_SCRATCH = [
    pltpu.VMEM((_HRW, CH), jnp.int32),
    pltpu.VMEM((_HRW, CH), jnp.int32),
    pltpu.VMEM((CH, D), jnp.float32),
    pltpu.VMEM((CH, D), jnp.float32),
    pltpu.VMEM_SHARED((NP, D), jnp.float32),
    pltpu.SemaphoreType.DMA,
    pltpu.SemaphoreType.DMA,
]


def _agg_body(table_hbm, src_hbm, dst_hbm, out_hbm,
              src_v, dst_v, rows0, rows1, acc, sem0, sem1):
    c = lax.axis_index("c")
    s = lax.axis_index("s")
    wid = s * NC + c

    @pl.loop(0, CH)
    def _(r):
        for k in range(D // 16):
            rows0[r, pl.ds(k * 16, 16)] = jnp.zeros((16,), jnp.float32)

    for j in range(NPT // CH):
        pltpu.sync_copy(rows0, acc.at[pl.ds(NPT * s + CH * j, CH)])
    plsc.subcore_barrier()

    for ph in range(2):
        base = wid * RW + ph * _HRW
        pltpu.sync_copy(src_hbm.at[pl.ds(base, _HRW)], src_v)
        pltpu.sync_copy(dst_hbm.at[pl.ds(base, _HRW)], dst_v)
        pltpu.async_copy(table_hbm.at[src_v.at[0]], rows0, sem0)

        @pl.loop(0, _HRW // 2)
        def _(k):
            pltpu.make_async_copy(table_hbm.at[src_v.at[0]], rows0, sem0).wait()
            pltpu.async_copy(table_hbm.at[src_v.at[2 * k + 1]], rows1, sem1)
            pltpu.sync_copy(rows0, acc.at[dst_v.at[2 * k]], add=True)
            pltpu.make_async_copy(table_hbm.at[src_v.at[0]], rows1, sem1).wait()

            @pl.when(k < _HRW // 2 - 1)
            def _():
                pltpu.async_copy(table_hbm.at[src_v.at[2 * k + 2]], rows0, sem0)

            pltpu.sync_copy(rows1, acc.at[dst_v.at[2 * k + 1]], add=True)

    plsc.subcore_barrier()
    pltpu.sync_copy(acc.at[pl.ds(NPT * s, NPT)], out_hbm.at[c, pl.ds(NPT * s, NPT)])


_agg_kernel = pl.kernel(
    _agg_body, out_type=_AGG_OUT, mesh=_mesh, scratch_types=_AGG_SCRATCH
)


BR = 2000
G = N // BR


def _sumhist_body(h_ref, o_ref):
    o_ref[...] = jnp.sum(h_ref[...], axis=0, keepdims=True) + 1.0


def _sumhist(hists):
    return pl.pallas_call(
        _sumhist_body,
        in_specs=[pl.BlockSpec((NW, NP), lambda: (0, 0))],
        out_specs=pl.BlockSpec((1, NP), lambda: (0, 0)),
        out_shape=jax.ShapeDtypeStruct((1, NP), jnp.float32),
    )(hists)


def _mm_relu_body(x_ref, w_ref, b_ref, o_ref):
    o_ref[...] = jnp.maximum(
        jnp.dot(x_ref[...], w_ref[...], preferred_element_type=jnp.float32)
        + b_ref[...],
        0.0,
    )


def _mm_relu(x, w, b):
    return pl.pallas_call(
        _mm_relu_body,
        grid=(G,),
        in_specs=[
            pl.BlockSpec((BR, D), lambda i: (i, 0)),
            pl.BlockSpec((D, D), lambda i: (0, 0)),
            pl.BlockSpec((1, D), lambda i: (0, 0)),
        ],
        out_specs=pl.BlockSpec((BR, D), lambda i: (i, 0)),
        out_shape=jax.ShapeDtypeStruct((N, D), jnp.float32),
    )(x, w, b.reshape(1, D))


def _pre_body(x_ref, wi_ref, b_ref, w_ref, d_ref, s_ref, dis_ref):
    h = jnp.maximum(
        jnp.dot(x_ref[...], wi_ref[...], preferred_element_type=jnp.float32)
        + b_ref[...],
        0.0,
    )
    dis = lax.rsqrt(d_ref[...])
    dis_ref[...] = dis
    s_ref[...] = (
        jnp.dot(h, w_ref[...], preferred_element_type=jnp.float32) * dis
    )


def _pre(x, wi, b, w, d):
    return pl.pallas_call(
        _pre_body,
        grid=(G,),
        in_specs=[
            pl.BlockSpec((BR, D), lambda i: (i, 0)),
            pl.BlockSpec((D, D), lambda i: (0, 0)),
            pl.BlockSpec((1, D), lambda i: (0, 0)),
            pl.BlockSpec((D, D), lambda i: (0, 0)),
            pl.BlockSpec((BR, 1), lambda i: (i, 0)),
        ],
        out_specs=[
            pl.BlockSpec((BR, D), lambda i: (i, 0)),
            pl.BlockSpec((BR, 1), lambda i: (i, 0)),
        ],
        out_shape=[
            jax.ShapeDtypeStruct((N, D), jnp.float32),
            jax.ShapeDtypeStruct((N, 1), jnp.float32),
        ],
    )(x, wi, b.reshape(1, D), w, d)


def _post_body(p0_ref, p1_ref, s_ref, dis_ref, b_ref, t_ref, sums_ref):
    t = (
        dis_ref[...] * (p0_ref[0] + p1_ref[0] + s_ref[...]) + b_ref[...]
    )
    t_ref[...] = t
    st = jnp.sum(t)
    st2 = jnp.sum(t * t)
    col = lax.broadcasted_iota(jnp.int32, (1, D), 1)
    sums_ref[0] = jnp.where(col == 0, st, 0.0) + jnp.where(col == 1, st2, 0.0)


def _post(p, s, dis, b):
    return pl.pallas_call(
        _post_body,
        grid=(G,),
        in_specs=[
            pl.BlockSpec((1, BR, D), lambda i: (0, i, 0)),
            pl.BlockSpec((1, BR, D), lambda i: (1, i, 0)),
            pl.BlockSpec((BR, D), lambda i: (i, 0)),
            pl.BlockSpec((BR, 1), lambda i: (i, 0)),
            pl.BlockSpec((1, D), lambda i: (0, 0)),
        ],
        out_specs=[
            pl.BlockSpec((BR, D), lambda i: (i, 0)),
            pl.BlockSpec((1, 1, D), lambda i: (i, 0, 0)),
        ],
        out_shape=[
            jax.ShapeDtypeStruct((N, D), jnp.float32),
            jax.ShapeDtypeStruct((G, 1, D), jnp.float32),
        ],
    )(p, p, s, dis, b.reshape(1, D))


_M = float(N * D)
_EPS = 1e-5


def _ln_mm_body(t_ref, sums_ref, dis_ref, g_ref, be_ref, w_ref, o_ref):
    sums = sums_ref[...]
    mean = jnp.sum(sums[:, 0, 0]) / _M
    var = jnp.sum(sums[:, 0, 1]) / _M - mean * mean
    inv = lax.rsqrt(var + _EPS)
    h = jnp.maximum((t_ref[...] - mean) * inv * g_ref[...] + be_ref[...], 0.0)
    o_ref[...] = (
        jnp.dot(h, w_ref[...], preferred_element_type=jnp.float32) * dis_ref[...]
    )


def _ln_mm(t, sums, dis, g, be, w):
    return pl.pallas_call(
        _ln_mm_body,
        grid=(G,),
        in_specs=[
            pl.BlockSpec((BR, D), lambda i: (i, 0)),
            pl.BlockSpec((G, 1, D), lambda i: (0, 0, 0)),
            pl.BlockSpec((BR, 1), lambda i: (i, 0)),
            pl.BlockSpec((1, D), lambda i: (0, 0)),
            pl.BlockSpec((1, D), lambda i: (0, 0)),
            pl.BlockSpec((D, D), lambda i: (0, 0)),
        ],
        out_specs=pl.BlockSpec((BR, D), lambda i: (i, 0)),
        out_shape=jax.ShapeDtypeStruct((N, D), jnp.float32),
    )(t, sums, dis, g.reshape(1, D), be.reshape(1, D), w)


def _final_body(t_ref, sums_ref, g_ref, be_ref, w_ref, b_ref, o_ref):
    sums = sums_ref[...]
    mean = jnp.sum(sums[:, 0, 0]) / _M
    var = jnp.sum(sums[:, 0, 1]) / _M - mean * mean
    inv = lax.rsqrt(var + _EPS)
    h = jnp.maximum((t_ref[...] - mean) * inv * g_ref[...] + be_ref[...], 0.0)
    logits = (
        jnp.dot(h, w_ref[...], preferred_element_type=jnp.float32) + b_ref[...]
    )
    m = jnp.max(logits, axis=-1, keepdims=True)
    e = jnp.exp(logits - m)
    o_ref[...] = e / jnp.sum(e, axis=-1, keepdims=True)


def _final(t, sums, g, be, w, b):
    return pl.pallas_call(
        _final_body,
        grid=(G,),
        in_specs=[
            pl.BlockSpec((BR, D), lambda i: (i, 0)),
            pl.BlockSpec((G, 1, D), lambda i: (0, 0, 0)),
            pl.BlockSpec((1, D), lambda i: (0, 0)),
            pl.BlockSpec((1, D), lambda i: (0, 0)),
            pl.BlockSpec((D, O), lambda i: (0, 0)),
            pl.BlockSpec((1, O), lambda i: (0, 0)),
        ],
        out_specs=pl.BlockSpec((BR, O), lambda i: (i, 0)),
        out_shape=jax.ShapeDtypeStruct((N, O), jnp.float32),
    )(t, sums, g.reshape(1, D), be.reshape(1, D), w, b.reshape(1, O))


def kernel(x, edge_index, W_in, b_in, W1, b1, g1, be1, W2, b2, g2, be2,
           W_out, b_out):
    npad = EP - E
    pad_src = (jnp.arange(npad, dtype=jnp.int32) * 991) % N
    pad_dst = N + (jnp.arange(npad, dtype=jnp.int32) % (NP - N))
    src1 = jnp.concatenate([edge_index[0], pad_src])
    dst1 = jnp.concatenate([edge_index[1], pad_dst])
    src2 = src1.reshape(ER, CH)
    dst2 = dst1.reshape(ER, CH)

    hists = _deg_kernel(dst1).reshape(NW, NP)
    degv = _sumhist(hists)
    d = degv.reshape(NP, 1)[:N]

    scaled1, dis = _pre(x, W_in, b_in, W1, d)

    pa = _agg_kernel(scaled1, src2, dst2)
    t1, s1 = _post(pa, scaled1, dis, b1)
    scaled2 = _ln_mm(t1, s1, dis, g1, be1, W2)

    pb = _agg_kernel(scaled2, src2, dst2)
    t2, s2 = _post(pb, scaled2, dis, b2)
    return _final(t2, s2, g2, be2, W_out, b_out)
